# Initial kernel scaffold; baseline (speedup 1.0000x reference)
import jax
import jax.numpy as jnp
from jax.experimental import pallas as pl


def kernel(adj_indices, adj_values, adj2_indices, adj2_values, emb_node, emb_attri, W_trans1, W_gc1, W_gc2):
    raise NotImplementedError("write your pallas kernel here")


# speedup vs baseline: 4.2406x; 4.2406x over previous
import functools

import jax
import jax.numpy as jnp
from jax import lax
from jax.experimental import pallas as pl
from jax.experimental.pallas import tpu as pltpu
from jax.experimental.pallas import tpu_sc as plsc

N_NODE = 8000
N_ATTRI = 2000
N_TOT = N_NODE + N_ATTRI
E_EDGES = 320000
F = 128
NSUB = 16
NW = 2 * NSUB
CH = 128
NCHT = E_EDGES // CH
NFULL = NCHT // NW
NREM = NCHT - NFULL * NW
NP = 10240
RPT = NP // NSUB
RB = 128
MROWS = 1000
CROWS = 1024


def _dense_body(x0_ref, w1_ref, wg1_ref, wg2_ref, x1_ref, h1_ref, h2_ref):
    x0 = x0_ref[...]
    dn = (((1,), (1,)), ((), ()))
    x1 = lax.dot_general(x0, w1_ref[...], dn, preferred_element_type=jnp.float32)
    x1_ref[...] = x1
    h1_ref[...] = lax.dot_general(x1, wg1_ref[...], dn,
                                  preferred_element_type=jnp.float32)
    h2_ref[...] = lax.dot_general(x1, wg2_ref[...], dn,
                                  preferred_element_type=jnp.float32)


_dense = pl.pallas_call(
    _dense_body,
    grid=(N_TOT // MROWS,),
    in_specs=[
        pl.BlockSpec((MROWS, F), lambda i: (i, 0)),
        pl.BlockSpec((F, F), lambda i: (0, 0)),
        pl.BlockSpec((F, F), lambda i: (0, 0)),
        pl.BlockSpec((F, F), lambda i: (0, 0)),
    ],
    out_specs=[
        pl.BlockSpec((MROWS, F), lambda i: (i, 0)),
        pl.BlockSpec((MROWS, F), lambda i: (i, 0)),
        pl.BlockSpec((MROWS, F), lambda i: (i, 0)),
    ],
    out_shape=[
        jax.ShapeDtypeStruct((N_TOT, F), jnp.float32),
        jax.ShapeDtypeStruct((N_TOT, F), jnp.float32),
        jax.ShapeDtypeStruct((N_TOT, F), jnp.float32),
    ],
)


def _combine_body(p1_ref, p2_ref, x2_ref, x3_ref):
    x2_ref[...] = jnp.maximum(p1_ref[0] + p1_ref[1], 0.0)
    x3_ref[...] = jnp.maximum(p2_ref[0] + p2_ref[1], 0.0)


_combine = pl.pallas_call(
    _combine_body,
    grid=(NP // CROWS,),
    in_specs=[
        pl.BlockSpec((2, CROWS, F), lambda i: (0, i, 0)),
        pl.BlockSpec((2, CROWS, F), lambda i: (0, i, 0)),
    ],
    out_specs=[
        pl.BlockSpec((CROWS, F), lambda i: (i, 0)),
        pl.BlockSpec((CROWS, F), lambda i: (i, 0)),
    ],
    out_shape=[
        jax.ShapeDtypeStruct((NP, F), jnp.float32),
        jax.ShapeDtypeStruct((NP, F), jnp.float32),
    ],
)


_mesh = plsc.VectorSubcoreMesh(core_axis_name="c", subcore_axis_name="s")


@functools.partial(
    pl.kernel,
    out_type=[
        jax.ShapeDtypeStruct((2, NP, F), jnp.float32),
        jax.ShapeDtypeStruct((2, NP, F), jnp.float32),
    ],
    mesh=_mesh,
    scratch_types=[
        pltpu.VMEM((2, CH), jnp.int32),
        pltpu.VMEM((CH,), jnp.float32),
        pltpu.VMEM((CH, F), jnp.float32),
        pltpu.VMEM((RB, F), jnp.float32),
        pltpu.VMEM_SHARED((NP, F), jnp.float32),
        pltpu.SemaphoreType.DMA,
    ],
)
def _sc_spmm(h1, dst1, src1, val1, h2, dst2, src2, val2,
             p1, p2,
             adj_v, val_v, rows_v, rb_v, acc, sem):
    c = lax.axis_index("c")
    s = lax.axis_index("s")
    w = s * 2 + c

    def _zbody(r, carry):
        for j in range(F // 16):
            rb_v[r, pl.ds(j * 16, 16)] = jnp.zeros((16,), jnp.float32)
        return carry

    lax.fori_loop(0, RB, _zbody, 0)

    gdn = lax.GatherDimensionNumbers(
        offset_dims=(), collapsed_slice_dims=(0,), start_index_map=(0,))

    def _phase(h_hbm, dst_hbm, src_hbm, vals_hbm, out_hbm):
        for k in range(RPT // RB):
            pltpu.sync_copy(rb_v, acc.at[pl.ds(s * RPT + k * RB, RB)])
        plsc.subcore_barrier()

        def _chunk(i, carry):
            base = (i * NW + w) * CH
            pltpu.sync_copy(dst_hbm.at[pl.ds(base, CH)], adj_v.at[0])
            pltpu.sync_copy(src_hbm.at[pl.ds(base, CH)], adj_v.at[1])
            pltpu.sync_copy(vals_hbm.at[pl.ds(base, CH)], val_v)
            pltpu.async_copy(h_hbm.at[adj_v.at[1]], rows_v, sem).wait()

            def _scale(g, carry2):
                vals16 = val_v[pl.ds(g * 16, 16)]
                for i2 in range(16):
                    vb = lax.gather(
                        vals16, jnp.full((16, 1), i2, jnp.int32), gdn, (1,),
                        mode=lax.GatherScatterMode.PROMISE_IN_BOUNDS)
                    for j in range(F // 16):
                        idx = (g * 16 + i2, pl.ds(j * 16, 16))
                        rows_v[idx] = rows_v[idx] * vb
                return carry2

            lax.fori_loop(0, CH // 16, _scale, 0)
            pltpu.sync_copy(rows_v, acc.at[adj_v.at[0]], add=True)
            return carry

        ntrip = NFULL + jnp.where(w < NREM, 1, 0)
        lax.fori_loop(0, ntrip, _chunk, 0)
        plsc.subcore_barrier()

        pltpu.sync_copy(acc.at[pl.ds(s * RPT, RPT)],
                        out_hbm.at[c].at[pl.ds(s * RPT, RPT)])
        plsc.subcore_barrier()

    _phase(h1, dst1, src1, val1, p1)
    _phase(h2, dst2, src2, val2, p2)


def kernel(adj_indices, adj_values, adj2_indices, adj2_values,
           emb_node, emb_attri, W_trans1, W_gc1, W_gc2):
    x0 = jnp.concatenate([emb_node, emb_attri], axis=0)
    x1, h1, h2 = _dense(x0, W_trans1, W_gc1, W_gc2)
    p1, p2 = _sc_spmm(
        h1, adj_indices[0], adj_indices[1], adj_values,
        h2, adj2_indices[0], adj2_indices[1], adj2_values)
    x2, x3 = _combine(p1, p2)
    return (x1, x2[:N_TOT], x3[:N_TOT])
